# Initial kernel scaffold; baseline (speedup 1.0000x reference)
import jax
import jax.numpy as jnp
from jax.experimental import pallas as pl


def kernel(x, Wq, Wk, Wv, Wo, W1, b1, W2, b2, g1, be1, g2, be2):
    raise NotImplementedError("write your pallas kernel here")


# speedup vs baseline: 20.1838x; 20.1838x over previous
import functools

import jax
import jax.numpy as jnp
import numpy as np
from jax.experimental import pallas as pl
from jax.experimental.pallas import tpu as pltpu

N = 512
D = 256
H = 4
DH = D // H


def _ln(x, g, b):
    mu = jnp.mean(x, axis=-1, keepdims=True)
    var = jnp.var(x, axis=-1, keepdims=True)
    return (x - mu) / jnp.sqrt(var + 1e-5) * g + b


def _gnn_kernel(x_ref, wq_ref, wk_ref, wv_ref, wo_ref, w1_ref, b1_ref,
                w2_ref, b2_ref, g1_ref, be1_ref, g2_ref, be2_ref, out_ref):
    z = x_ref[...]
    q = jnp.dot(z, wq_ref[...], preferred_element_type=jnp.float32)
    k = jnp.dot(z, wk_ref[...], preferred_element_type=jnp.float32)
    v = jnp.dot(z, wv_ref[...], preferred_element_type=jnp.float32)

    scale = np.float32(1.0 / np.sqrt(DH))
    aggs = []
    for h in range(H):
        sl = slice(h * DH, (h + 1) * DH)
        qh = q[:, sl]
        kh = k[:, sl]
        vh = v[:, sl]
        e = jnp.dot(qh, kh.T, preferred_element_type=jnp.float32) * scale
        m = jnp.max(e, axis=1, keepdims=True)
        ex = jnp.exp(e - m)
        ssum = jnp.sum(ex, axis=1, keepdims=True)
        alpha = ex / (ssum + 1e-9)
        aggs.append(jnp.dot(alpha, vh, preferred_element_type=jnp.float32))
    agg = jnp.concatenate(aggs, axis=1)

    out = jnp.dot(agg, wo_ref[...], preferred_element_type=jnp.float32)
    z1 = _ln(z + out, g1_ref[...], be1_ref[...])
    hmid = jax.nn.relu(
        jnp.dot(z1, w1_ref[...], preferred_element_type=jnp.float32)
        + b1_ref[...])
    hout = jnp.dot(hmid, w2_ref[...], preferred_element_type=jnp.float32) \
        + b2_ref[...]
    out_ref[...] = _ln(z1 + hout, g2_ref[...], be2_ref[...])


@functools.partial(jax.jit, static_argnames=())
def _run(x, Wq, Wk, Wv, Wo, W1, b1, W2, b2, g1, be1, g2, be2):
    vecs = [b1.reshape(1, D), b2.reshape(1, D), g1.reshape(1, D),
            be1.reshape(1, D), g2.reshape(1, D), be2.reshape(1, D)]
    z2 = pl.pallas_call(
        _gnn_kernel,
        out_shape=jax.ShapeDtypeStruct((N, D), jnp.float32),
    )(x, Wq, Wk, Wv, Wo, W1, vecs[0], W2, vecs[1],
      vecs[2], vecs[3], vecs[4], vecs[5])
    return (x, z2)


def kernel(x, Wq, Wk, Wv, Wo, W1, b1, W2, b2, g1, be1, g2, be2):
    return _run(x, Wq, Wk, Wv, Wo, W1, b1, W2, b2, g1, be1, g2, be2)
